# baseline (speedup 1.0000x reference)
import functools

import jax
import jax.numpy as jnp
from jax import lax
from jax.experimental import pallas as pl
from jax.experimental.pallas import tpu as pltpu
from jax.experimental.pallas import tpu_sc as plsc

NC = 2
NSC = 16
NW = NC * NSC
L = 16
SSP = 104


def _sc_mesh():
    return plsc.VectorSubcoreMesh(
        core_axis_name="c", subcore_axis_name="s",
        num_cores=NC, num_subcores=NSC)


def _tc_project(feat, W1):
    n, d = feat.shape
    e = W1.shape[0]
    blk = 1000
    assert n % blk == 0

    def body(x_ref, w_ref, o_ref):
        o_ref[...] = lax.dot_general(
            x_ref[...], w_ref[...], (((1,), (1,)), ((), ())),
            preferred_element_type=jnp.float32)

    return pl.pallas_call(
        body,
        grid=(n // blk,),
        in_specs=[
            pl.BlockSpec((blk, d), lambda i: (i, 0)),
            pl.BlockSpec((e, d), lambda i: (0, 0)),
        ],
        out_specs=pl.BlockSpec((blk, e), lambda i: (i, 0)),
        out_shape=jax.ShapeDtypeStruct((n, e), jnp.float32),
    )(feat, W1)


def _tc_pad_neigh(neigh_idx, width):
    n, s = neigh_idx.shape
    blk = 1000
    assert n % blk == 0

    def body(x_ref, o_ref):
        o_ref[...] = jnp.concatenate(
            [x_ref[...], jnp.zeros((blk, width - s), jnp.int32)], axis=1)

    return pl.pallas_call(
        body,
        grid=(n // blk,),
        in_specs=[pl.BlockSpec((blk, s), lambda i: (i, 0))],
        out_specs=pl.BlockSpec((blk, width), lambda i: (i, 0)),
        out_shape=jax.ShapeDtypeStruct((n, width), jnp.int32),
    )(neigh_idx)


def _tc_head(P, W2, W_cls, scale):
    b2, e = P.shape
    c = W_cls.shape[0]

    def body(p_ref, w2_ref, wc_ref, o_ref):
        wc2 = lax.dot_general(
            wc_ref[...], w2_ref[...], (((1,), (0,)), ((), ())),
            preferred_element_type=jnp.float32)
        o_ref[...] = scale * lax.dot_general(
            p_ref[...], wc2, (((1,), (1,)), ((), ())),
            preferred_element_type=jnp.float32)

    return pl.pallas_call(
        body, out_shape=jax.ShapeDtypeStruct((b2, c), jnp.float32),
    )(P, W2, W_cls)


def _sc_indices(neigh_pad, nodes_flat, s):
    n, emb = neigh_pad.shape
    nslot = nodes_flat.shape[0]
    slots_w = nslot // NW
    ss = s * s
    lvl1 = slots_w * s
    nchunk = lvl1 // slots_w

    @functools.partial(
        pl.kernel,
        out_type=jax.ShapeDtypeStruct((NW, slots_w * SSP), jnp.int32),
        mesh=_sc_mesh(),
        compiler_params=pltpu.CompilerParams(needs_layout_passes=False),
        scratch_types=[
            pltpu.VMEM((slots_w,), jnp.int32),
            pltpu.VMEM((lvl1,), jnp.int32),
            pltpu.VMEM((2, slots_w, emb), jnp.int32),
            pltpu.VMEM((slots_w * SSP,), jnp.int32),
            pltpu.SemaphoreType.DMA,
            pltpu.SemaphoreType.DMA,
        ],
    )
    def idx_kernel(ni_hbm, nodes_hbm, out_hbm,
                   nodes_v, nb2f, nbd2, nbf, semc0, semc1):
        semc = (semc0, semc1)
        wid = lax.axis_index("s") * NC + lax.axis_index("c")
        base_slot = wid * slots_w

        pltpu.sync_copy(nodes_hbm.at[pl.ds(base_slot, slots_w)], nodes_v)
        pltpu.async_copy(ni_hbm.at[nodes_v], nbd2.at[0], semc[0]).wait()

        iota = lax.iota(jnp.int32, L)
        zero16 = jnp.zeros((L,), jnp.int32)

        def div_s(x):
            assert s == 10
            return (x * 6554) >> 16

        def flat1(t, carry):
            k = t * L + iota
            row = div_s(k)
            col = k - row * s
            v = plsc.load_gather(nbd2, [zero16, row, col])
            nb2f[pl.ds(pl.multiple_of(t * L, L), L)] = v
            return carry
        lax.fori_loop(0, lvl1 // L, flat1, 0)

        def fillpad(t, carry):
            r = t * L + iota
            for dc in range(SSP - ss):
                plsc.store_scatter(nbf, [r * SSP + (ss + dc)], zero16)
            return carry
        lax.fori_loop(0, slots_w // L, fillpad, 0)

        def scat_chunk(c, p):
            def scat(t, carry2):
                k = t * L + iota
                j = div_s(k)
                sp = k - j * s
                m = c * slots_w + j
                i = div_s(m)
                s1 = m - i * s
                v = plsc.load_gather(nbd2, [zero16 + p, j, sp])
                plsc.store_scatter(nbf, [i * SSP + s1 * s + sp], v)
                return carry2
            lax.fori_loop(0, lvl1 // L, scat, 0)

        def lvl2(cc, carry):
            cps = []
            for p in range(2):
                c = cc * 2 + p
                idx = nb2f.at[pl.ds(pl.multiple_of(c * slots_w, 8), slots_w)]
                cps.append(
                    pltpu.async_copy(ni_hbm.at[idx], nbd2.at[p], semc[p]))
            for p in range(2):
                cps[p].wait()
                scat_chunk(cc * 2 + p, p)
            return carry
        lax.fori_loop(0, nchunk // 2, lvl2, 0)

        pltpu.sync_copy(nbf, out_hbm.at[wid])

    return idx_kernel(neigh_pad, nodes_flat)


def _sc_main(G, nbf_all, s):
    n, emb = G.shape
    nw, wlen = nbf_all.shape
    slots_w = wlen // SSP
    bw = slots_w // 2
    nb = NW * bw
    nv = emb // L
    ndeep = 4

    @functools.partial(
        pl.kernel,
        out_type=jax.ShapeDtypeStruct((nb, emb), jnp.float32),
        mesh=_sc_mesh(),
        compiler_params=pltpu.CompilerParams(needs_layout_passes=False),
        scratch_types=[
            pltpu.VMEM((slots_w * SSP,), jnp.int32),
            pltpu.VMEM((ndeep, SSP, emb), jnp.float32),
            pltpu.VMEM((bw, emb), jnp.float32),
            pltpu.SemaphoreType.DMA,
            pltpu.SemaphoreType.DMA,
            pltpu.SemaphoreType.DMA,
            pltpu.SemaphoreType.DMA,
        ],
    )
    def main_kernel(g_hbm, nbf_hbm, out_hbm,
                    nbf, grow, out_v, sem0, sem1, sem2, sem3):
        sems = (sem0, sem1, sem2, sem3)
        wid = lax.axis_index("s") * NC + lax.axis_index("c")
        pltpu.sync_copy(nbf_hbm.at[wid], nbf)

        def per_bb(bb, carry):
            cps = []
            for j in range(ndeep):
                slot4 = bb * 4 + j
                idx3 = nbf.at[pl.ds(pl.multiple_of(slot4 * SSP, 8), SSP)]
                cps.append(
                    pltpu.async_copy(g_hbm.at[idx3], grow.at[j], sems[j]))
            for bpair in range(2):
                b = bb * 2 + bpair
                acc = [jnp.zeros((L,), jnp.float32) for _ in range(nv)]
                for e in range(2):
                    j = bpair * 2 + e
                    cps[j].wait()

                    def per_s(si, acc_c):
                        part = [jnp.zeros((L,), jnp.float32)
                                for _ in range(nv)]
                        for t in range(s):
                            r = si * s + t
                            for v in range(nv):
                                part[v] = part[v] + grow[j, r,
                                                         pl.ds(v * L, L)]
                        return [a + jnp.maximum(p, 0.0)
                                for a, p in zip(acc_c, part)]
                    acc = lax.fori_loop(0, s, per_s, acc)
                for v in range(nv):
                    out_v[b, pl.ds(v * L, L)] = acc[v]
            return carry
        lax.fori_loop(0, bw // 2, per_bb, 0)

        pltpu.sync_copy(out_v, out_hbm.at[pl.ds(wid * bw, bw)])

    return main_kernel(G, nbf_all)


def kernel(feat, W1, W2, W_cls, neigh_idx, nodes):
    s = neigh_idx.shape[1]
    ni_pad = _tc_pad_neigh(neigh_idx.astype(jnp.int32), feat.shape[1])
    nbf_all = _sc_indices(ni_pad, nodes.reshape(-1).astype(jnp.int32), s)
    G = _tc_project(feat, W1)
    P = _sc_main(G, nbf_all, s)
    return _tc_head(P, W2, W_cls, 1.0 / (s * s * 2))
